# Initial kernel scaffold; baseline (speedup 1.0000x reference)
import jax
import jax.numpy as jnp
from jax.experimental import pallas as pl


def kernel(x, edge_index, batch, c1_Wl, c1_bl, c1_Wr, n1_w, n1_b, n1_ms, c2_Wl, c2_bl, c2_Wr, n2_w, n2_b, n2_ms, c3_Wl, c3_bl, c3_Wr, n3_w, n3_b, n3_ms, c4_Wl, c4_bl, c4_Wr, n4_w, n4_b, n4_ms, c5_Wl, c5_bl, c5_Wr):
    raise NotImplementedError("write your pallas kernel here")


# speedup vs baseline: 2.8389x; 2.8389x over previous
import functools
import jax
import jax.numpy as jnp
from jax import lax
from jax.experimental import pallas as pl
from jax.experimental.pallas import tpu as pltpu
from jax.experimental.pallas import tpu_sc as plsc

N = 10000
E = 320000
NG = 8
IN_F = 128
HID = 512
OUT_F = 121

NC = 2
NS = 16
LANES = 128
ROWS_TOTAL = 2560
EP = ROWS_TOTAL * LANES
NP = 10112
ROWS_PER_TILE_1 = ROWS_TOTAL // (NC * NS)
ROWS_PER_TILE_4 = ROWS_TOTAL // NS
NODE_ROWS_PER_TILE = NP // NS

@functools.cache
def _sc_kernels():
  mesh = plsc.VectorSubcoreMesh(
      core_axis_name="c", subcore_axis_name="s", num_cores=NC, num_subcores=NS)
  sc_agg1 = pl.kernel(
      _sc_agg1_body,
      out_type=jax.ShapeDtypeStruct((NC, NP, IN_F), jnp.float32),
      mesh=mesh,
      scratch_types=[
          pltpu.VMEM((IC, LANES), jnp.int32),
          pltpu.VMEM((IC, LANES), jnp.int32),
          pltpu.VMEM((LANES, IN_F), jnp.float32),
          pltpu.VMEM_SHARED((NP, IN_F), jnp.float32),
          pltpu.SemaphoreType.DMA,
      ],
  )
  sc_agg4 = pl.kernel(
      _sc_agg4_body,
      out_type=jax.ShapeDtypeStruct((4, NP, 128), jnp.float32),
      mesh=mesh,
      scratch_types=[
          pltpu.VMEM((IC, LANES), jnp.int32),
          pltpu.VMEM((IC, LANES), jnp.int32),
          pltpu.VMEM((LANES, 128), jnp.float32),
          pltpu.VMEM_SHARED((NP, 128), jnp.float32),
          pltpu.SemaphoreType.DMA,
      ],
  )
  return sc_agg1, sc_agg4


IC = 16


def _sc_agg1_body(table_hbm, src_hbm, dst_hbm, zeros_hbm, out_hbm,
                  sv, dv, rows, acc, sem):
  cid = lax.axis_index("c")
  sid = lax.axis_index("s")
  wid = cid * NS + sid
  nz = NODE_ROWS_PER_TILE
  pltpu.sync_copy(zeros_hbm.at[pl.ds(sid * nz, nz)], acc.at[pl.ds(sid * nz, nz)])
  nb = ROWS_PER_TILE_1
  base = wid * nb
  plsc.subcore_barrier()

  def outer(oc, carry):
    off = base + oc * IC
    pltpu.sync_copy(src_hbm.at[pl.ds(off, IC)], sv)
    pltpu.sync_copy(dst_hbm.at[pl.ds(off, IC)], dv)

    def body(j, c2):
      pltpu.async_copy(table_hbm.at[sv.at[j]], rows, sem).wait()
      pltpu.sync_copy(rows, acc.at[dv.at[j]], add=True)
      return c2

    lax.fori_loop(0, IC, body, 0)
    return carry

  lax.fori_loop(0, nb // IC, outer, 0)
  plsc.subcore_barrier()
  pltpu.sync_copy(acc.at[pl.ds(sid * nz, nz)],
                  out_hbm.at[cid, pl.ds(sid * nz, nz)])


def _sc_agg4_body(table_hbm, srcadj_hbm, dst_hbm, zeros_hbm, out_hbm,
                  sv, dv, rows, acc, sem):
  cid = lax.axis_index("c")
  sid = lax.axis_index("s")
  nz = NODE_ROWS_PER_TILE
  nb = ROWS_PER_TILE_4
  base = sid * nb
  for j in range(2):
    chunk = cid * 2 + j
    pltpu.sync_copy(zeros_hbm.at[pl.ds(sid * nz, nz)],
                    acc.at[pl.ds(sid * nz, nz)])
    plsc.subcore_barrier()

    def outer(oc, carry):
      off = base + oc * IC
      pltpu.sync_copy(srcadj_hbm.at[chunk, pl.ds(off, IC)], sv)
      pltpu.sync_copy(dst_hbm.at[pl.ds(off, IC)], dv)

      def body(jj, c2):
        pltpu.async_copy(table_hbm.at[sv.at[jj]], rows, sem).wait()
        pltpu.sync_copy(rows, acc.at[dv.at[jj]], add=True)
        return c2

      lax.fori_loop(0, IC, body, 0)
      return carry

    lax.fori_loop(0, nb // IC, outer, 0)
    plsc.subcore_barrier()
    pltpu.sync_copy(acc.at[pl.ds(sid * nz, nz)],
                    out_hbm.at[chunk, pl.ds(sid * nz, nz)])
    plsc.subcore_barrier()


NB = 1000
NBLK = N // NB


def _onehot(batch_blk):
  g = lax.broadcasted_iota(jnp.int32, (1, NG), 1)
  return (batch_blk == g).astype(jnp.float32)


def _moments_update(i, z, oh, s1_ref, s2_ref, cnt_ref):
  @pl.when(i == 0)
  def _():
    s1_ref[...] = jnp.zeros_like(s1_ref)
    s2_ref[...] = jnp.zeros_like(s2_ref)
    cnt_ref[...] = jnp.zeros_like(cnt_ref)

  dn = (((0,), (0,)), ((), ()))
  s1_ref[...] += lax.dot_general(oh, z, dn, preferred_element_type=jnp.float32, precision=lax.Precision.HIGHEST)
  s2_ref[...] += lax.dot_general(oh, z * z, dn, preferred_element_type=jnp.float32, precision=lax.Precision.HIGHEST)
  cnt_ref[...] += jnp.sum(oh, axis=0)[:, None]


def _alpha_beta(i, ms, w, b, s1_ref, s2_ref, cnt_ref, alpha_ref, beta_ref):
  @pl.when(i == NBLK - 1)
  def _():
    cnt = jnp.maximum(cnt_ref[:, 0:1], 1.0)
    m = s1_ref[...] / cnt
    var = s2_ref[...] / cnt - (2.0 * ms - ms * ms) * m * m
    std = jnp.sqrt(var + 1e-5)
    alpha = w / std
    alpha_ref[...] = alpha
    beta_ref[...] = b - alpha * m * ms


def _tc_a1_body(aggp, degp, x, batch_blk, Wl, bl, Wr, ms, w, b,
                z_ref, alpha_ref, beta_ref, s1_ref, s2_ref, cnt_ref):
  i = pl.program_id(0)
  deg = degp[0, :, 0:1] + degp[1, :, 0:1]
  invd = 1.0 / jnp.maximum(deg, 1.0)
  agg = (aggp[0] + aggp[1]) * invd
  z = (jnp.dot(agg, Wl[...], preferred_element_type=jnp.float32, precision=lax.Precision.HIGHEST)
       + jnp.dot(x[...], Wr[...], preferred_element_type=jnp.float32, precision=lax.Precision.HIGHEST)
       + bl[...])
  z_ref[...] = z
  oh = _onehot(batch_blk[...])
  _moments_update(i, z, oh, s1_ref, s2_ref, cnt_ref)
  _alpha_beta(i, ms[...], w[...], b[...], s1_ref, s2_ref, cnt_ref,
              alpha_ref, beta_ref)


def _tc_a_body(aggc, degp, hc, batch_blk, Wl, bl, Wr, ms, w, b,
               z_ref, alpha_ref, beta_ref, s1_ref, s2_ref, cnt_ref):
  i = pl.program_id(0)
  deg = degp[0, :, 0:1] + degp[1, :, 0:1]
  invd = 1.0 / jnp.maximum(deg, 1.0)
  z = bl[...]
  for c in range(4):
    z = z + jnp.dot(aggc[c] * invd, Wl[c], preferred_element_type=jnp.float32, precision=lax.Precision.HIGHEST)
    z = z + jnp.dot(hc[c], Wr[c], preferred_element_type=jnp.float32, precision=lax.Precision.HIGHEST)
  z_ref[...] = z
  oh = _onehot(batch_blk[...])
  _moments_update(i, z, oh, s1_ref, s2_ref, cnt_ref)
  _alpha_beta(i, ms[...], w[...], b[...], s1_ref, s2_ref, cnt_ref,
              alpha_ref, beta_ref)


def _tc_b_body(z, alpha, beta, batch_blk, hc_ref):
  oh = _onehot(batch_blk[...])
  a_rows = jnp.dot(oh, alpha[...], preferred_element_type=jnp.float32, precision=lax.Precision.HIGHEST)
  b_rows = jnp.dot(oh, beta[...], preferred_element_type=jnp.float32, precision=lax.Precision.HIGHEST)
  h = jnp.maximum(a_rows * z[...] + b_rows, 0.0)
  for c in range(4):
    hc_ref[c] = h[:, c * 128:(c + 1) * 128]


def _tc_b4_body(z, alpha, beta, batch_blk, Wl5, hc_ref, y5_ref):
  oh = _onehot(batch_blk[...])
  a_rows = jnp.dot(oh, alpha[...], preferred_element_type=jnp.float32, precision=lax.Precision.HIGHEST)
  b_rows = jnp.dot(oh, beta[...], preferred_element_type=jnp.float32, precision=lax.Precision.HIGHEST)
  h = jnp.maximum(a_rows * z[...] + b_rows, 0.0)
  for c in range(4):
    hc_ref[c] = h[:, c * 128:(c + 1) * 128]
  y5_ref[...] = jnp.dot(h, Wl5[...], preferred_element_type=jnp.float32, precision=lax.Precision.HIGHEST)


def _tc_final_body(y5p, degp, hc, bl, Wr, out_ref):
  deg = degp[0, :, 0:1] + degp[1, :, 0:1]
  invd = 1.0 / jnp.maximum(deg, 1.0)
  out = (y5p[0] + y5p[1]) * invd + bl[...]
  for c in range(4):
    out = out + jnp.dot(hc[c], Wr[c], preferred_element_type=jnp.float32, precision=lax.Precision.HIGHEST)
  out_ref[...] = out


def _row_spec(shape_prefix, block):
  pass


_spec_aggp = pl.BlockSpec((NC, NB, 128), lambda i: (0, i, 0))
_spec_degp = pl.BlockSpec((NC, NB, 128), lambda i: (0, i, 0))
_spec_x = pl.BlockSpec((NB, IN_F), lambda i: (i, 0))
_spec_batch = pl.BlockSpec((NB, 1), lambda i: (i, 0))
_spec_z = pl.BlockSpec((NB, HID), lambda i: (i, 0))
_spec_full2 = lambda a, bdim: pl.BlockSpec((a, bdim), lambda i: (0, 0))
_spec_hc = pl.BlockSpec((4, NB, 128), lambda i: (0, i, 0))
_spec_Wc = pl.BlockSpec((4, 128, HID), lambda i: (0, 0, 0))

_stat_scratch = [
    pltpu.VMEM((NG, HID), jnp.float32),
    pltpu.VMEM((NG, HID), jnp.float32),
    pltpu.VMEM((NG, 128), jnp.float32),
]
_ab_out_specs = [
    pl.BlockSpec((NB, HID), lambda i: (i, 0)),
    pl.BlockSpec((NG, HID), lambda i: (0, 0)),
    pl.BlockSpec((NG, HID), lambda i: (0, 0)),
]
_ab_out_shape = [
    jax.ShapeDtypeStruct((N, HID), jnp.float32),
    jax.ShapeDtypeStruct((NG, HID), jnp.float32),
    jax.ShapeDtypeStruct((NG, HID), jnp.float32),
]

_tc_a1 = pl.pallas_call(
    _tc_a1_body,
    grid=(NBLK,),
    in_specs=[
        _spec_aggp, _spec_degp, _spec_x, _spec_batch,
        _spec_full2(IN_F, HID), _spec_full2(1, HID), _spec_full2(IN_F, HID),
        _spec_full2(1, HID), _spec_full2(1, HID), _spec_full2(1, HID),
    ],
    out_specs=_ab_out_specs,
    out_shape=_ab_out_shape,
    scratch_shapes=_stat_scratch,
)

_tc_a = pl.pallas_call(
    _tc_a_body,
    grid=(NBLK,),
    in_specs=[
        pl.BlockSpec((4, NB, 128), lambda i: (0, i, 0)), _spec_degp, _spec_hc,
        _spec_batch,
        _spec_Wc, _spec_full2(1, HID), _spec_Wc,
        _spec_full2(1, HID), _spec_full2(1, HID), _spec_full2(1, HID),
    ],
    out_specs=_ab_out_specs,
    out_shape=_ab_out_shape,
    scratch_shapes=_stat_scratch,
)

_tc_b = pl.pallas_call(
    _tc_b_body,
    grid=(NBLK,),
    in_specs=[
        _spec_z, _spec_full2(NG, HID), _spec_full2(NG, HID), _spec_batch,
    ],
    out_specs=_spec_hc,
    out_shape=jax.ShapeDtypeStruct((4, N, 128), jnp.float32),
)

_tc_b4 = pl.pallas_call(
    _tc_b4_body,
    grid=(NBLK,),
    in_specs=[
        _spec_z, _spec_full2(NG, HID), _spec_full2(NG, HID), _spec_batch,
        _spec_full2(HID, 128),
    ],
    out_specs=[_spec_hc, pl.BlockSpec((NB, 128), lambda i: (i, 0))],
    out_shape=[
        jax.ShapeDtypeStruct((4, N, 128), jnp.float32),
        jax.ShapeDtypeStruct((N, 128), jnp.float32),
    ],
)

_tc_final = pl.pallas_call(
    _tc_final_body,
    grid=(NBLK,),
    in_specs=[
        _spec_aggp, _spec_degp, _spec_hc,
        pl.BlockSpec((1, 128), lambda i: (0, 0)),
        pl.BlockSpec((4, 128, 128), lambda i: (0, 0, 0)),
    ],
    out_specs=pl.BlockSpec((NB, 128), lambda i: (i, 0)),
    out_shape=jax.ShapeDtypeStruct((N, 128), jnp.float32),
)


def kernel(x, edge_index, batch,
           c1_Wl, c1_bl, c1_Wr, n1_w, n1_b, n1_ms,
           c2_Wl, c2_bl, c2_Wr, n2_w, n2_b, n2_ms,
           c3_Wl, c3_bl, c3_Wr, n3_w, n3_b, n3_ms,
           c4_Wl, c4_bl, c4_Wr, n4_w, n4_b, n4_ms,
           c5_Wl, c5_bl, c5_Wr):
  f32 = jnp.float32
  _sc_agg1, _sc_agg4 = _sc_kernels()
  src = edge_index[0]
  dst = edge_index[1]
  src_p = jnp.concatenate([src, jnp.zeros((EP - E,), jnp.int32)])
  dst_p = jnp.concatenate([dst, jnp.full((EP - E,), N, jnp.int32)])
  src2d = src_p.reshape(ROWS_TOTAL, LANES)
  dst2d = dst_p.reshape(ROWS_TOTAL, LANES)
  srcadj = (src_p[None, :] + (jnp.arange(4, dtype=jnp.int32) * N)[:, None]
            ).reshape(4, ROWS_TOTAL, LANES)
  zeros128 = jnp.zeros((NP, 128), f32)
  batch2d = batch.reshape(N, 1)

  degp = _sc_agg1(jnp.ones((N, 128), f32), src2d, dst2d, zeros128)

  def row(v):
    return v.reshape(1, -1)

  p1 = _sc_agg1(x, src2d, dst2d, zeros128)
  z, al, be = _tc_a1(p1, degp, x, batch2d, c1_Wl, row(c1_bl), c1_Wr,
                     row(n1_ms), row(n1_w), row(n1_b))
  hc = _tc_b(z, al, be, batch2d)

  for (Wl, bl, Wr, msv, wv, bv) in (
      (c2_Wl, c2_bl, c2_Wr, n2_ms, n2_w, n2_b),
      (c3_Wl, c3_bl, c3_Wr, n3_ms, n3_w, n3_b),
  ):
    table = hc.reshape(4 * N, 128)
    s4 = _sc_agg4(table, srcadj, dst2d, zeros128)
    z, al, be = _tc_a(s4, degp, hc, batch2d,
                      Wl.reshape(4, 128, HID), row(bl), Wr.reshape(4, 128, HID),
                      row(msv), row(wv), row(bv))
    hc = _tc_b(z, al, be, batch2d)

  table = hc.reshape(4 * N, 128)
  s4 = _sc_agg4(table, srcadj, dst2d, zeros128)
  z, al, be = _tc_a(s4, degp, hc, batch2d,
                    c4_Wl.reshape(4, 128, HID), row(c4_bl),
                    c4_Wr.reshape(4, 128, HID),
                    row(n4_ms), row(n4_w), row(n4_b))
  Wl5p = jnp.pad(c5_Wl, ((0, 0), (0, 128 - OUT_F)))
  hc, y5 = _tc_b4(z, al, be, batch2d, Wl5p)

  p5 = _sc_agg1(y5, src2d, dst2d, zeros128)
  Wr5p = jnp.pad(c5_Wr, ((0, 0), (0, 128 - OUT_F))).reshape(4, 128, 128)
  bl5p = jnp.pad(c5_bl, (0, 128 - OUT_F)).reshape(1, 128)
  out = _tc_final(p5, degp, hc, bl5p, Wr5p)
  return out[:, :OUT_F]


# speedup vs baseline: 3.2143x; 1.1323x over previous
import functools
import jax
import jax.numpy as jnp
from jax import lax
from jax.experimental import pallas as pl
from jax.experimental.pallas import tpu as pltpu
from jax.experimental.pallas import tpu_sc as plsc

N = 10000
E = 320000
NG = 8
IN_F = 128
HID = 512
OUT_F = 121

NC = 2
NS = 16
LANES = 128
ROWS_TOTAL = 2560
EP = ROWS_TOTAL * LANES
NP = 10112
ROWS_PER_TILE_1 = ROWS_TOTAL // (NC * NS)
ROWS_PER_TILE_4 = ROWS_TOTAL // NS
NODE_ROWS_PER_TILE = NP // NS

@functools.cache
def _sc_kernels():
  mesh = plsc.VectorSubcoreMesh(
      core_axis_name="c", subcore_axis_name="s", num_cores=NC, num_subcores=NS)
  agg_scratch = [
      pltpu.VMEM((IC, LANES), jnp.int32),
      pltpu.VMEM((IC, LANES), jnp.int32),
      pltpu.VMEM((LANES, 128), jnp.float32),
      pltpu.VMEM((LANES, 128), jnp.float32),
      pltpu.VMEM_SHARED((NP, 128), jnp.float32),
      pltpu.SemaphoreType.DMA,
      pltpu.SemaphoreType.DMA,
  ]
  sc_agg1 = pl.kernel(
      _sc_agg1_body,
      out_type=jax.ShapeDtypeStruct((NC, NP, IN_F), jnp.float32),
      mesh=mesh,
      scratch_types=agg_scratch,
  )
  sc_agg4 = pl.kernel(
      _sc_agg4_body,
      out_type=jax.ShapeDtypeStruct((4, NP, 128), jnp.float32),
      mesh=mesh,
      scratch_types=agg_scratch,
  )
  sc_deg = pl.kernel(
      _sc_deg_body,
      out_type=jax.ShapeDtypeStruct((NC, NP, 128), jnp.float32),
      mesh=mesh,
      scratch_types=[
          pltpu.VMEM((IC, LANES), jnp.int32),
          pltpu.VMEM((LANES, 128), jnp.float32),
          pltpu.VMEM_SHARED((NP, 128), jnp.float32),
          pltpu.SemaphoreType.DMA,
      ],
  )
  return sc_agg1, sc_agg4, sc_deg


IC = 16


def _gather_scatter_chunk(table_hbm, sv, dv, rows0, rows1, acc, gsem, ssem):
  bufs = (rows0, rows1)
  gds = [pltpu.async_copy(table_hbm.at[sv.at[0]], bufs[0], gsem)]
  sds = []
  for r in range(IC):
    gds[r].wait()
    sds.append(pltpu.async_copy(bufs[r % 2], acc.at[dv.at[r]], ssem, add=True))
    if r >= 1:
      sds[r - 1].wait()
    if r + 1 < IC:
      gds.append(pltpu.async_copy(table_hbm.at[sv.at[r + 1]],
                                  bufs[(r + 1) % 2], gsem))
  sds[IC - 1].wait()


def _sc_agg1_body(table_hbm, src_hbm, dst_hbm, zeros_hbm, out_hbm,
                  sv, dv, rows0, rows1, acc, gsem, ssem):
  cid = lax.axis_index("c")
  sid = lax.axis_index("s")
  wid = cid * NS + sid
  nz = NODE_ROWS_PER_TILE
  pltpu.sync_copy(zeros_hbm.at[pl.ds(sid * nz, nz)], acc.at[pl.ds(sid * nz, nz)])
  nb = ROWS_PER_TILE_1
  base = wid * nb
  plsc.subcore_barrier()

  def outer(oc, carry):
    off = base + oc * IC
    pltpu.sync_copy(src_hbm.at[pl.ds(off, IC)], sv)
    pltpu.sync_copy(dst_hbm.at[pl.ds(off, IC)], dv)
    _gather_scatter_chunk(table_hbm, sv, dv, rows0, rows1, acc, gsem, ssem)
    return carry

  lax.fori_loop(0, nb // IC, outer, 0)
  plsc.subcore_barrier()
  pltpu.sync_copy(acc.at[pl.ds(sid * nz, nz)],
                  out_hbm.at[cid, pl.ds(sid * nz, nz)])


def _sc_deg_body(dst_hbm, zeros_hbm, ones_hbm, out_hbm, dv, ones_v, acc, ssem):
  cid = lax.axis_index("c")
  sid = lax.axis_index("s")
  wid = cid * NS + sid
  nz = NODE_ROWS_PER_TILE
  pltpu.sync_copy(zeros_hbm.at[pl.ds(sid * nz, nz)], acc.at[pl.ds(sid * nz, nz)])
  pltpu.sync_copy(ones_hbm, ones_v)
  nb = ROWS_PER_TILE_1
  base = wid * nb
  plsc.subcore_barrier()

  def outer(oc, carry):
    off = base + oc * IC
    pltpu.sync_copy(dst_hbm.at[pl.ds(off, IC)], dv)
    sds = [pltpu.async_copy(ones_v, acc.at[dv.at[r]], ssem, add=True)
           for r in range(IC)]
    for d in sds:
      d.wait()
    return carry

  lax.fori_loop(0, nb // IC, outer, 0)
  plsc.subcore_barrier()
  pltpu.sync_copy(acc.at[pl.ds(sid * nz, nz)],
                  out_hbm.at[cid, pl.ds(sid * nz, nz)])


def _sc_agg4_body(table_hbm, srcadj_hbm, dst_hbm, zeros_hbm, out_hbm,
                  sv, dv, rows0, rows1, acc, gsem, ssem):
  cid = lax.axis_index("c")
  sid = lax.axis_index("s")
  nz = NODE_ROWS_PER_TILE
  nb = ROWS_PER_TILE_4
  base = sid * nb
  for j in range(2):
    chunk = cid * 2 + j
    pltpu.sync_copy(zeros_hbm.at[pl.ds(sid * nz, nz)],
                    acc.at[pl.ds(sid * nz, nz)])
    plsc.subcore_barrier()

    def outer(oc, carry):
      off = base + oc * IC
      pltpu.sync_copy(srcadj_hbm.at[chunk, pl.ds(off, IC)], sv)
      pltpu.sync_copy(dst_hbm.at[pl.ds(off, IC)], dv)
      _gather_scatter_chunk(table_hbm, sv, dv, rows0, rows1, acc, gsem, ssem)
      return carry

    lax.fori_loop(0, nb // IC, outer, 0)
    plsc.subcore_barrier()
    pltpu.sync_copy(acc.at[pl.ds(sid * nz, nz)],
                    out_hbm.at[chunk, pl.ds(sid * nz, nz)])
    plsc.subcore_barrier()


NB = 1000
NBLK = N // NB


def _onehot(batch_blk):
  g = lax.broadcasted_iota(jnp.int32, (1, NG), 1)
  return (batch_blk == g).astype(jnp.float32)


def _moments_update(i, z, oh, s1_ref, s2_ref, cnt_ref):
  @pl.when(i == 0)
  def _():
    s1_ref[...] = jnp.zeros_like(s1_ref)
    s2_ref[...] = jnp.zeros_like(s2_ref)
    cnt_ref[...] = jnp.zeros_like(cnt_ref)

  dn = (((0,), (0,)), ((), ()))
  s1_ref[...] += lax.dot_general(oh, z, dn, preferred_element_type=jnp.float32, precision=lax.Precision.HIGHEST)
  s2_ref[...] += lax.dot_general(oh, z * z, dn, preferred_element_type=jnp.float32, precision=lax.Precision.HIGHEST)
  cnt_ref[...] += jnp.sum(oh, axis=0)[:, None]


def _alpha_beta(i, ms, w, b, s1_ref, s2_ref, cnt_ref, alpha_ref, beta_ref):
  @pl.when(i == NBLK - 1)
  def _():
    cnt = jnp.maximum(cnt_ref[:, 0:1], 1.0)
    m = s1_ref[...] / cnt
    var = s2_ref[...] / cnt - (2.0 * ms - ms * ms) * m * m
    std = jnp.sqrt(var + 1e-5)
    alpha = w / std
    alpha_ref[...] = alpha
    beta_ref[...] = b - alpha * m * ms


def _tc_a1_body(aggp, degp, x, batch_blk, Wl, bl, Wr, ms, w, b,
                z_ref, alpha_ref, beta_ref, s1_ref, s2_ref, cnt_ref):
  i = pl.program_id(0)
  deg = degp[0, :, 0:1] + degp[1, :, 0:1]
  invd = 1.0 / jnp.maximum(deg, 1.0)
  agg = (aggp[0] + aggp[1]) * invd
  z = (jnp.dot(agg, Wl[...], preferred_element_type=jnp.float32, precision=lax.Precision.HIGHEST)
       + jnp.dot(x[...], Wr[...], preferred_element_type=jnp.float32, precision=lax.Precision.HIGHEST)
       + bl[...])
  z_ref[...] = z
  oh = _onehot(batch_blk[...])
  _moments_update(i, z, oh, s1_ref, s2_ref, cnt_ref)
  _alpha_beta(i, ms[...], w[...], b[...], s1_ref, s2_ref, cnt_ref,
              alpha_ref, beta_ref)


def _tc_a_body(aggc, degp, hc, batch_blk, Wl, bl, Wr, ms, w, b,
               z_ref, alpha_ref, beta_ref, s1_ref, s2_ref, cnt_ref):
  i = pl.program_id(0)
  deg = degp[0, :, 0:1] + degp[1, :, 0:1]
  invd = 1.0 / jnp.maximum(deg, 1.0)
  z = bl[...]
  for c in range(4):
    z = z + jnp.dot(aggc[c] * invd, Wl[c], preferred_element_type=jnp.float32, precision=lax.Precision.HIGHEST)
    z = z + jnp.dot(hc[c], Wr[c], preferred_element_type=jnp.float32, precision=lax.Precision.HIGHEST)
  z_ref[...] = z
  oh = _onehot(batch_blk[...])
  _moments_update(i, z, oh, s1_ref, s2_ref, cnt_ref)
  _alpha_beta(i, ms[...], w[...], b[...], s1_ref, s2_ref, cnt_ref,
              alpha_ref, beta_ref)


def _tc_b_body(z, alpha, beta, batch_blk, hc_ref):
  oh = _onehot(batch_blk[...])
  a_rows = jnp.dot(oh, alpha[...], preferred_element_type=jnp.float32, precision=lax.Precision.HIGHEST)
  b_rows = jnp.dot(oh, beta[...], preferred_element_type=jnp.float32, precision=lax.Precision.HIGHEST)
  h = jnp.maximum(a_rows * z[...] + b_rows, 0.0)
  for c in range(4):
    hc_ref[c] = h[:, c * 128:(c + 1) * 128]


def _tc_b4_body(z, alpha, beta, batch_blk, Wl5, hc_ref, y5_ref):
  oh = _onehot(batch_blk[...])
  a_rows = jnp.dot(oh, alpha[...], preferred_element_type=jnp.float32, precision=lax.Precision.HIGHEST)
  b_rows = jnp.dot(oh, beta[...], preferred_element_type=jnp.float32, precision=lax.Precision.HIGHEST)
  h = jnp.maximum(a_rows * z[...] + b_rows, 0.0)
  for c in range(4):
    hc_ref[c] = h[:, c * 128:(c + 1) * 128]
  y5_ref[...] = jnp.dot(h, Wl5[...], preferred_element_type=jnp.float32, precision=lax.Precision.HIGHEST)


def _tc_final_body(y5p, degp, hc, bl, Wr, out_ref):
  deg = degp[0, :, 0:1] + degp[1, :, 0:1]
  invd = 1.0 / jnp.maximum(deg, 1.0)
  out = (y5p[0] + y5p[1]) * invd + bl[...]
  for c in range(4):
    out = out + jnp.dot(hc[c], Wr[c], preferred_element_type=jnp.float32, precision=lax.Precision.HIGHEST)
  out_ref[...] = out


def _row_spec(shape_prefix, block):
  pass


_spec_aggp = pl.BlockSpec((NC, NB, 128), lambda i: (0, i, 0))
_spec_degp = pl.BlockSpec((NC, NB, 128), lambda i: (0, i, 0))
_spec_x = pl.BlockSpec((NB, IN_F), lambda i: (i, 0))
_spec_batch = pl.BlockSpec((NB, 1), lambda i: (i, 0))
_spec_z = pl.BlockSpec((NB, HID), lambda i: (i, 0))
_spec_full2 = lambda a, bdim: pl.BlockSpec((a, bdim), lambda i: (0, 0))
_spec_hc = pl.BlockSpec((4, NB, 128), lambda i: (0, i, 0))
_spec_Wc = pl.BlockSpec((4, 128, HID), lambda i: (0, 0, 0))

_stat_scratch = [
    pltpu.VMEM((NG, HID), jnp.float32),
    pltpu.VMEM((NG, HID), jnp.float32),
    pltpu.VMEM((NG, 128), jnp.float32),
]
_ab_out_specs = [
    pl.BlockSpec((NB, HID), lambda i: (i, 0)),
    pl.BlockSpec((NG, HID), lambda i: (0, 0)),
    pl.BlockSpec((NG, HID), lambda i: (0, 0)),
]
_ab_out_shape = [
    jax.ShapeDtypeStruct((N, HID), jnp.float32),
    jax.ShapeDtypeStruct((NG, HID), jnp.float32),
    jax.ShapeDtypeStruct((NG, HID), jnp.float32),
]

_tc_a1 = pl.pallas_call(
    _tc_a1_body,
    grid=(NBLK,),
    in_specs=[
        _spec_aggp, _spec_degp, _spec_x, _spec_batch,
        _spec_full2(IN_F, HID), _spec_full2(1, HID), _spec_full2(IN_F, HID),
        _spec_full2(1, HID), _spec_full2(1, HID), _spec_full2(1, HID),
    ],
    out_specs=_ab_out_specs,
    out_shape=_ab_out_shape,
    scratch_shapes=_stat_scratch,
)

_tc_a = pl.pallas_call(
    _tc_a_body,
    grid=(NBLK,),
    in_specs=[
        pl.BlockSpec((4, NB, 128), lambda i: (0, i, 0)), _spec_degp, _spec_hc,
        _spec_batch,
        _spec_Wc, _spec_full2(1, HID), _spec_Wc,
        _spec_full2(1, HID), _spec_full2(1, HID), _spec_full2(1, HID),
    ],
    out_specs=_ab_out_specs,
    out_shape=_ab_out_shape,
    scratch_shapes=_stat_scratch,
)

_tc_b = pl.pallas_call(
    _tc_b_body,
    grid=(NBLK,),
    in_specs=[
        _spec_z, _spec_full2(NG, HID), _spec_full2(NG, HID), _spec_batch,
    ],
    out_specs=_spec_hc,
    out_shape=jax.ShapeDtypeStruct((4, N, 128), jnp.float32),
)

_tc_b4 = pl.pallas_call(
    _tc_b4_body,
    grid=(NBLK,),
    in_specs=[
        _spec_z, _spec_full2(NG, HID), _spec_full2(NG, HID), _spec_batch,
        _spec_full2(HID, 128),
    ],
    out_specs=[_spec_hc, pl.BlockSpec((NB, 128), lambda i: (i, 0))],
    out_shape=[
        jax.ShapeDtypeStruct((4, N, 128), jnp.float32),
        jax.ShapeDtypeStruct((N, 128), jnp.float32),
    ],
)

_tc_final = pl.pallas_call(
    _tc_final_body,
    grid=(NBLK,),
    in_specs=[
        _spec_aggp, _spec_degp, _spec_hc,
        pl.BlockSpec((1, 128), lambda i: (0, 0)),
        pl.BlockSpec((4, 128, 128), lambda i: (0, 0, 0)),
    ],
    out_specs=pl.BlockSpec((NB, 128), lambda i: (i, 0)),
    out_shape=jax.ShapeDtypeStruct((N, 128), jnp.float32),
)


def kernel(x, edge_index, batch,
           c1_Wl, c1_bl, c1_Wr, n1_w, n1_b, n1_ms,
           c2_Wl, c2_bl, c2_Wr, n2_w, n2_b, n2_ms,
           c3_Wl, c3_bl, c3_Wr, n3_w, n3_b, n3_ms,
           c4_Wl, c4_bl, c4_Wr, n4_w, n4_b, n4_ms,
           c5_Wl, c5_bl, c5_Wr):
  f32 = jnp.float32
  _sc_agg1, _sc_agg4, _sc_deg = _sc_kernels()
  src = edge_index[0]
  dst = edge_index[1]
  src_p = jnp.concatenate([src, jnp.zeros((EP - E,), jnp.int32)])
  dst_p = jnp.concatenate([dst, jnp.full((EP - E,), N, jnp.int32)])
  src2d = src_p.reshape(ROWS_TOTAL, LANES)
  dst2d = dst_p.reshape(ROWS_TOTAL, LANES)
  srcadj = (src_p[None, :] + (jnp.arange(4, dtype=jnp.int32) * N)[:, None]
            ).reshape(4, ROWS_TOTAL, LANES)
  zeros128 = jnp.zeros((NP, 128), f32)
  batch2d = batch.reshape(N, 1)

  degp = _sc_deg(dst2d, zeros128, jnp.ones((LANES, 128), f32))

  def row(v):
    return v.reshape(1, -1)

  p1 = _sc_agg1(x, src2d, dst2d, zeros128)
  z, al, be = _tc_a1(p1, degp, x, batch2d, c1_Wl, row(c1_bl), c1_Wr,
                     row(n1_ms), row(n1_w), row(n1_b))
  hc = _tc_b(z, al, be, batch2d)

  for (Wl, bl, Wr, msv, wv, bv) in (
      (c2_Wl, c2_bl, c2_Wr, n2_ms, n2_w, n2_b),
      (c3_Wl, c3_bl, c3_Wr, n3_ms, n3_w, n3_b),
  ):
    table = hc.reshape(4 * N, 128)
    s4 = _sc_agg4(table, srcadj, dst2d, zeros128)
    z, al, be = _tc_a(s4, degp, hc, batch2d,
                      Wl.reshape(4, 128, HID), row(bl), Wr.reshape(4, 128, HID),
                      row(msv), row(wv), row(bv))
    hc = _tc_b(z, al, be, batch2d)

  table = hc.reshape(4 * N, 128)
  s4 = _sc_agg4(table, srcadj, dst2d, zeros128)
  z, al, be = _tc_a(s4, degp, hc, batch2d,
                    c4_Wl.reshape(4, 128, HID), row(c4_bl),
                    c4_Wr.reshape(4, 128, HID),
                    row(n4_ms), row(n4_w), row(n4_b))
  Wl5p = jnp.pad(c5_Wl, ((0, 0), (0, 128 - OUT_F)))
  hc, y5 = _tc_b4(z, al, be, batch2d, Wl5p)

  p5 = _sc_agg1(y5, src2d, dst2d, zeros128)
  Wr5p = jnp.pad(c5_Wr, ((0, 0), (0, 128 - OUT_F))).reshape(4, 128, 128)
  bl5p = jnp.pad(c5_bl, (0, 128 - OUT_F)).reshape(1, 128)
  out = _tc_final(p5, degp, hc, bl5p, Wr5p)
  return out[:, :OUT_F]


# speedup vs baseline: 3.2160x; 1.0005x over previous
import functools
import jax
import jax.numpy as jnp
from jax import lax
from jax.experimental import pallas as pl
from jax.experimental.pallas import tpu as pltpu
from jax.experimental.pallas import tpu_sc as plsc

N = 10000
E = 320000
NG = 8
IN_F = 128
HID = 512
OUT_F = 121

NC = 2
NS = 16
LANES = 128
ROWS_TOTAL = 2560
EP = ROWS_TOTAL * LANES
NP = 10112
ROWS_PER_TILE_1 = ROWS_TOTAL // (NC * NS)
ROWS_PER_TILE_4 = ROWS_TOTAL // NS
NODE_ROWS_PER_TILE = NP // NS

@functools.cache
def _sc_kernels():
  mesh = plsc.VectorSubcoreMesh(
      core_axis_name="c", subcore_axis_name="s", num_cores=NC, num_subcores=NS)
  agg_scratch = [
      pltpu.VMEM((IC, LANES), jnp.int32),
      pltpu.VMEM((IC, LANES), jnp.int32),
      pltpu.VMEM((LANES, 128), jnp.float32),
      pltpu.VMEM((LANES, 128), jnp.float32),
      pltpu.VMEM_SHARED((NP, 128), jnp.float32),
      pltpu.SemaphoreType.DMA,
      pltpu.SemaphoreType.DMA,
  ]
  sc_agg1 = pl.kernel(
      _sc_agg1_body,
      out_type=jax.ShapeDtypeStruct((NC, NP, IN_F), jnp.float32),
      mesh=mesh,
      scratch_types=agg_scratch,
  )
  sc_agg4 = pl.kernel(
      _sc_agg4_body,
      out_type=jax.ShapeDtypeStruct((4, NP, 128), jnp.float32),
      mesh=mesh,
      scratch_types=agg_scratch,
  )
  sc_deg = pl.kernel(
      _sc_deg_body,
      out_type=jax.ShapeDtypeStruct((NC, NP, 128), jnp.float32),
      mesh=mesh,
      scratch_types=[
          pltpu.VMEM((IC, LANES), jnp.int32),
          pltpu.VMEM((LANES, 128), jnp.float32),
          pltpu.VMEM_SHARED((NP, 128), jnp.float32),
          pltpu.SemaphoreType.DMA,
      ],
  )
  return sc_agg1, sc_agg4, sc_deg


IC = 16


def _gather_scatter_chunk(table_hbm, sv, dv, rows0, rows1, acc, gsem, ssem):
  bufs = (rows0, rows1)
  gds = [pltpu.async_copy(table_hbm.at[sv.at[0]], bufs[0], gsem)]
  sds = []
  for r in range(IC):
    gds[r].wait()
    sds.append(pltpu.async_copy(bufs[r % 2], acc.at[dv.at[r]], ssem, add=True))
    if r >= 1:
      sds[r - 1].wait()
    if r + 1 < IC:
      gds.append(pltpu.async_copy(table_hbm.at[sv.at[r + 1]],
                                  bufs[(r + 1) % 2], gsem))
  sds[IC - 1].wait()


def _sc_agg1_body(table_hbm, src_hbm, dst_hbm, zeros_hbm, out_hbm,
                  sv, dv, rows0, rows1, acc, gsem, ssem):
  cid = lax.axis_index("c")
  sid = lax.axis_index("s")
  wid = cid * NS + sid
  nz = NODE_ROWS_PER_TILE
  pltpu.sync_copy(zeros_hbm.at[pl.ds(sid * nz, nz)], acc.at[pl.ds(sid * nz, nz)])
  nb = ROWS_PER_TILE_1
  base = wid * nb
  plsc.subcore_barrier()

  def outer(oc, carry):
    off = base + oc * IC
    pltpu.sync_copy(src_hbm.at[pl.ds(off, IC)], sv)
    pltpu.sync_copy(dst_hbm.at[pl.ds(off, IC)], dv)
    _gather_scatter_chunk(table_hbm, sv, dv, rows0, rows1, acc, gsem, ssem)
    return carry

  lax.fori_loop(0, nb // IC, outer, 0)
  plsc.subcore_barrier()
  pltpu.sync_copy(acc.at[pl.ds(sid * nz, nz)],
                  out_hbm.at[cid, pl.ds(sid * nz, nz)])


def _sc_deg_body(dst_hbm, zeros_hbm, ones_hbm, out_hbm, dv, ones_v, acc, ssem):
  cid = lax.axis_index("c")
  sid = lax.axis_index("s")
  wid = cid * NS + sid
  nz = NODE_ROWS_PER_TILE
  pltpu.sync_copy(zeros_hbm.at[pl.ds(sid * nz, nz)], acc.at[pl.ds(sid * nz, nz)])
  pltpu.sync_copy(ones_hbm, ones_v)
  nb = ROWS_PER_TILE_1
  base = wid * nb
  plsc.subcore_barrier()

  def outer(oc, carry):
    off = base + oc * IC
    pltpu.sync_copy(dst_hbm.at[pl.ds(off, IC)], dv)
    sds = [pltpu.async_copy(ones_v, acc.at[dv.at[r]], ssem, add=True)
           for r in range(IC)]
    for d in sds:
      d.wait()
    return carry

  lax.fori_loop(0, nb // IC, outer, 0)
  plsc.subcore_barrier()
  pltpu.sync_copy(acc.at[pl.ds(sid * nz, nz)],
                  out_hbm.at[cid, pl.ds(sid * nz, nz)])


def _sc_agg4_body(table_hbm, srcadj_hbm, dst_hbm, zeros_hbm, out_hbm,
                  sv, dv, rows0, rows1, acc, gsem, ssem):
  cid = lax.axis_index("c")
  sid = lax.axis_index("s")
  nz = NODE_ROWS_PER_TILE
  nb = ROWS_PER_TILE_4
  base = sid * nb
  for j in range(2):
    chunk = cid * 2 + j
    pltpu.sync_copy(zeros_hbm.at[pl.ds(sid * nz, nz)],
                    acc.at[pl.ds(sid * nz, nz)])
    plsc.subcore_barrier()

    def outer(oc, carry):
      off = base + oc * IC
      pltpu.sync_copy(srcadj_hbm.at[chunk, pl.ds(off, IC)], sv)
      pltpu.sync_copy(dst_hbm.at[pl.ds(off, IC)], dv)
      _gather_scatter_chunk(table_hbm, sv, dv, rows0, rows1, acc, gsem, ssem)
      return carry

    lax.fori_loop(0, nb // IC, outer, 0)
    plsc.subcore_barrier()
    pltpu.sync_copy(acc.at[pl.ds(sid * nz, nz)],
                    out_hbm.at[chunk, pl.ds(sid * nz, nz)])
    plsc.subcore_barrier()


NB = 1000
NBLK = N // NB


def _onehot(batch_blk):
  g = lax.broadcasted_iota(jnp.int32, (1, NG), 1)
  return (batch_blk == g).astype(jnp.float32)


def _moments_update(i, z, oh, s1_ref, s2_ref, cnt_ref):
  @pl.when(i == 0)
  def _():
    s1_ref[...] = jnp.zeros_like(s1_ref)
    s2_ref[...] = jnp.zeros_like(s2_ref)
    cnt_ref[...] = jnp.zeros_like(cnt_ref)

  dn = (((0,), (0,)), ((), ()))
  s1_ref[...] += lax.dot_general(oh, z, dn, preferred_element_type=jnp.float32, precision=lax.Precision.HIGHEST)
  s2_ref[...] += lax.dot_general(oh, z * z, dn, preferred_element_type=jnp.float32, precision=lax.Precision.HIGHEST)
  cnt_ref[...] += jnp.sum(oh, axis=0)[:, None]


def _alpha_beta(i, ms, w, b, s1_ref, s2_ref, cnt_ref, alpha_ref, beta_ref):
  @pl.when(i == NBLK - 1)
  def _():
    cnt = jnp.maximum(cnt_ref[:, 0:1], 1.0)
    m = s1_ref[...] / cnt
    var = s2_ref[...] / cnt - (2.0 * ms - ms * ms) * m * m
    std = jnp.sqrt(var + 1e-5)
    alpha = w / std
    alpha_ref[...] = alpha
    beta_ref[...] = b - alpha * m * ms


def _tc_a1_body(aggp, degp, x, batch_blk, Wl, bl, Wr, ms, w, b,
                z_ref, alpha_ref, beta_ref, s1_ref, s2_ref, cnt_ref):
  i = pl.program_id(0)
  deg = degp[0, :, 0:1] + degp[1, :, 0:1]
  invd = 1.0 / jnp.maximum(deg, 1.0)
  agg = (aggp[0] + aggp[1]) * invd
  z = (jnp.dot(agg, Wl[...], preferred_element_type=jnp.float32, precision=lax.Precision.HIGHEST)
       + jnp.dot(x[...], Wr[...], preferred_element_type=jnp.float32, precision=lax.Precision.HIGHEST)
       + bl[...])
  z_ref[...] = z
  oh = _onehot(batch_blk[...])
  _moments_update(i, z, oh, s1_ref, s2_ref, cnt_ref)
  _alpha_beta(i, ms[...], w[...], b[...], s1_ref, s2_ref, cnt_ref,
              alpha_ref, beta_ref)


def _tc_a_body(aggc, degp, hc, batch_blk, Wl, bl, Wr, ms, w, b,
               z_ref, alpha_ref, beta_ref, s1_ref, s2_ref, cnt_ref):
  i = pl.program_id(0)
  deg = degp[0, :, 0:1] + degp[1, :, 0:1]
  invd = 1.0 / jnp.maximum(deg, 1.0)
  z = bl[...]
  for c in range(4):
    z = z + jnp.dot(aggc[c] * invd, Wl[c], preferred_element_type=jnp.float32, precision=lax.Precision.HIGHEST)
    z = z + jnp.dot(hc[c], Wr[c], preferred_element_type=jnp.float32, precision=lax.Precision.HIGHEST)
  z_ref[...] = z
  oh = _onehot(batch_blk[...])
  _moments_update(i, z, oh, s1_ref, s2_ref, cnt_ref)
  _alpha_beta(i, ms[...], w[...], b[...], s1_ref, s2_ref, cnt_ref,
              alpha_ref, beta_ref)


def _tc_b_body(z, alpha, beta, batch_blk, hc_ref):
  oh = _onehot(batch_blk[...])
  a_rows = jnp.dot(oh, alpha[...], preferred_element_type=jnp.float32, precision=lax.Precision.HIGHEST)
  b_rows = jnp.dot(oh, beta[...], preferred_element_type=jnp.float32, precision=lax.Precision.HIGHEST)
  h = jnp.maximum(a_rows * z[...] + b_rows, 0.0)
  for c in range(4):
    hc_ref[c] = h[:, c * 128:(c + 1) * 128]


def _tc_b4_body(z, alpha, beta, batch_blk, Wl5, hc_ref, y5_ref):
  oh = _onehot(batch_blk[...])
  a_rows = jnp.dot(oh, alpha[...], preferred_element_type=jnp.float32, precision=lax.Precision.HIGHEST)
  b_rows = jnp.dot(oh, beta[...], preferred_element_type=jnp.float32, precision=lax.Precision.HIGHEST)
  h = jnp.maximum(a_rows * z[...] + b_rows, 0.0)
  for c in range(4):
    hc_ref[c] = h[:, c * 128:(c + 1) * 128]
  y5_ref[...] = jnp.dot(h, Wl5[...], preferred_element_type=jnp.float32, precision=lax.Precision.HIGHEST)


def _tc_final_body(y5p, degp, hc, bl, Wr, out_ref):
  deg = degp[0, :, 0:1] + degp[1, :, 0:1]
  invd = 1.0 / jnp.maximum(deg, 1.0)
  out = (y5p[0] + y5p[1]) * invd + bl[...]
  for c in range(4):
    out = out + jnp.dot(hc[c], Wr[c], preferred_element_type=jnp.float32, precision=lax.Precision.HIGHEST)
  out_ref[...] = out


def _row_spec(shape_prefix, block):
  pass


_spec_aggp = pl.BlockSpec((NC, NB, 128), lambda i: (0, i, 0))
_spec_degp = pl.BlockSpec((NC, NB, 128), lambda i: (0, i, 0))
_spec_x = pl.BlockSpec((NB, IN_F), lambda i: (i, 0))
_spec_batch = pl.BlockSpec((NB, 1), lambda i: (i, 0))
_spec_z = pl.BlockSpec((NB, HID), lambda i: (i, 0))
_spec_full2 = lambda a, bdim: pl.BlockSpec((a, bdim), lambda i: (0, 0))
_spec_hc = pl.BlockSpec((4, NB, 128), lambda i: (0, i, 0))
_spec_Wc = pl.BlockSpec((4, 128, HID), lambda i: (0, 0, 0))

_stat_scratch = [
    pltpu.VMEM((NG, HID), jnp.float32),
    pltpu.VMEM((NG, HID), jnp.float32),
    pltpu.VMEM((NG, 128), jnp.float32),
]
_ab_out_specs = [
    pl.BlockSpec((NB, HID), lambda i: (i, 0)),
    pl.BlockSpec((NG, HID), lambda i: (0, 0)),
    pl.BlockSpec((NG, HID), lambda i: (0, 0)),
]
_ab_out_shape = [
    jax.ShapeDtypeStruct((N, HID), jnp.float32),
    jax.ShapeDtypeStruct((NG, HID), jnp.float32),
    jax.ShapeDtypeStruct((NG, HID), jnp.float32),
]

_tc_a1 = pl.pallas_call(
    _tc_a1_body,
    grid=(NBLK,),
    in_specs=[
        _spec_aggp, _spec_degp, _spec_x, _spec_batch,
        _spec_full2(IN_F, HID), _spec_full2(1, HID), _spec_full2(IN_F, HID),
        _spec_full2(1, HID), _spec_full2(1, HID), _spec_full2(1, HID),
    ],
    out_specs=_ab_out_specs,
    out_shape=_ab_out_shape,
    scratch_shapes=_stat_scratch,
)

_tc_a = pl.pallas_call(
    _tc_a_body,
    grid=(NBLK,),
    in_specs=[
        pl.BlockSpec((4, NB, 128), lambda i: (0, i, 0)), _spec_degp, _spec_hc,
        _spec_batch,
        _spec_Wc, _spec_full2(1, HID), _spec_Wc,
        _spec_full2(1, HID), _spec_full2(1, HID), _spec_full2(1, HID),
    ],
    out_specs=_ab_out_specs,
    out_shape=_ab_out_shape,
    scratch_shapes=_stat_scratch,
)

_tc_b = pl.pallas_call(
    _tc_b_body,
    grid=(NBLK,),
    in_specs=[
        _spec_z, _spec_full2(NG, HID), _spec_full2(NG, HID), _spec_batch,
    ],
    out_specs=_spec_hc,
    out_shape=jax.ShapeDtypeStruct((4, N, 128), jnp.float32),
)

_tc_b4 = pl.pallas_call(
    _tc_b4_body,
    grid=(NBLK,),
    in_specs=[
        _spec_z, _spec_full2(NG, HID), _spec_full2(NG, HID), _spec_batch,
        _spec_full2(HID, 128),
    ],
    out_specs=[_spec_hc, pl.BlockSpec((NB, 128), lambda i: (i, 0))],
    out_shape=[
        jax.ShapeDtypeStruct((4, N, 128), jnp.float32),
        jax.ShapeDtypeStruct((N, 128), jnp.float32),
    ],
)

_tc_final = pl.pallas_call(
    _tc_final_body,
    grid=(NBLK,),
    in_specs=[
        _spec_aggp, _spec_degp, _spec_hc,
        pl.BlockSpec((1, 128), lambda i: (0, 0)),
        pl.BlockSpec((4, 128, 128), lambda i: (0, 0, 0)),
    ],
    out_specs=pl.BlockSpec((NB, 128), lambda i: (i, 0)),
    out_shape=jax.ShapeDtypeStruct((N, 128), jnp.float32),
)


def kernel(x, edge_index, batch,
           c1_Wl, c1_bl, c1_Wr, n1_w, n1_b, n1_ms,
           c2_Wl, c2_bl, c2_Wr, n2_w, n2_b, n2_ms,
           c3_Wl, c3_bl, c3_Wr, n3_w, n3_b, n3_ms,
           c4_Wl, c4_bl, c4_Wr, n4_w, n4_b, n4_ms,
           c5_Wl, c5_bl, c5_Wr):
  f32 = jnp.float32
  _sc_agg1, _sc_agg4, _sc_deg = _sc_kernels()
  src = edge_index[0]
  dst = edge_index[1]
  src_p = jnp.concatenate([src, jnp.zeros((EP - E,), jnp.int32)])
  pad_dst = N + jnp.arange(EP - E, dtype=jnp.int32) % (NP - N)
  dst_p = jnp.concatenate([dst, pad_dst])
  src2d = src_p.reshape(ROWS_TOTAL, LANES)
  dst2d = dst_p.reshape(ROWS_TOTAL, LANES)
  srcadj = (src_p[None, :] + (jnp.arange(4, dtype=jnp.int32) * N)[:, None]
            ).reshape(4, ROWS_TOTAL, LANES)
  zeros128 = jnp.zeros((NP, 128), f32)
  batch2d = batch.reshape(N, 1)

  degp = _sc_deg(dst2d, zeros128, jnp.ones((LANES, 128), f32))

  def row(v):
    return v.reshape(1, -1)

  p1 = _sc_agg1(x, src2d, dst2d, zeros128)
  z, al, be = _tc_a1(p1, degp, x, batch2d, c1_Wl, row(c1_bl), c1_Wr,
                     row(n1_ms), row(n1_w), row(n1_b))
  hc = _tc_b(z, al, be, batch2d)

  for (Wl, bl, Wr, msv, wv, bv) in (
      (c2_Wl, c2_bl, c2_Wr, n2_ms, n2_w, n2_b),
      (c3_Wl, c3_bl, c3_Wr, n3_ms, n3_w, n3_b),
  ):
    table = hc.reshape(4 * N, 128)
    s4 = _sc_agg4(table, srcadj, dst2d, zeros128)
    z, al, be = _tc_a(s4, degp, hc, batch2d,
                      Wl.reshape(4, 128, HID), row(bl), Wr.reshape(4, 128, HID),
                      row(msv), row(wv), row(bv))
    hc = _tc_b(z, al, be, batch2d)

  table = hc.reshape(4 * N, 128)
  s4 = _sc_agg4(table, srcadj, dst2d, zeros128)
  z, al, be = _tc_a(s4, degp, hc, batch2d,
                    c4_Wl.reshape(4, 128, HID), row(c4_bl),
                    c4_Wr.reshape(4, 128, HID),
                    row(n4_ms), row(n4_w), row(n4_b))
  Wl5p = jnp.pad(c5_Wl, ((0, 0), (0, 128 - OUT_F)))
  hc, y5 = _tc_b4(z, al, be, batch2d, Wl5p)

  p5 = _sc_agg1(y5, src2d, dst2d, zeros128)
  Wr5p = jnp.pad(c5_Wr, ((0, 0), (0, 128 - OUT_F))).reshape(4, 128, 128)
  bl5p = jnp.pad(c5_bl, (0, 128 - OUT_F)).reshape(1, 128)
  out = _tc_final(p5, degp, hc, bl5p, Wr5p)
  return out[:, :OUT_F]


# speedup vs baseline: 3.2369x; 1.0065x over previous
import functools
import jax
import jax.numpy as jnp
from jax import lax
from jax.experimental import pallas as pl
from jax.experimental.pallas import tpu as pltpu
from jax.experimental.pallas import tpu_sc as plsc

N = 10000
E = 320000
NG = 8
IN_F = 128
HID = 512
OUT_F = 121

NC = 2
NS = 16
LANES = 128
ROWS_TOTAL = 2560
EP = ROWS_TOTAL * LANES
NP = 10112
ROWS_PER_TILE_1 = ROWS_TOTAL // (NC * NS)
ROWS_PER_TILE_4 = ROWS_TOTAL // NS
NODE_ROWS_PER_TILE = NP // NS

@functools.cache
def _sc_kernels():
  mesh = plsc.VectorSubcoreMesh(
      core_axis_name="c", subcore_axis_name="s", num_cores=NC, num_subcores=NS)
  def agg_scratch(ic):
    return [
        pltpu.VMEM((ic, LANES), jnp.int32),
        pltpu.VMEM((ic, LANES), jnp.int32),
        pltpu.VMEM((LANES, 128), jnp.float32),
        pltpu.VMEM((LANES, 128), jnp.float32),
        pltpu.VMEM_SHARED((NP, 128), jnp.float32),
        pltpu.SemaphoreType.DMA,
        pltpu.SemaphoreType.DMA,
    ]
  sc_agg1 = pl.kernel(
      _sc_agg1_body,
      out_type=jax.ShapeDtypeStruct((NC, NP, IN_F), jnp.float32),
      mesh=mesh,
      scratch_types=agg_scratch(IC1),
  )
  sc_agg4 = pl.kernel(
      _sc_agg4_body,
      out_type=jax.ShapeDtypeStruct((4, NP, 128), jnp.float32),
      mesh=mesh,
      scratch_types=agg_scratch(IC4),
  )
  sc_deg = pl.kernel(
      _sc_deg_body,
      out_type=jax.ShapeDtypeStruct((NC, NP, 128), jnp.float32),
      mesh=mesh,
      scratch_types=[
          pltpu.VMEM((IC1, LANES), jnp.int32),
          pltpu.VMEM((LANES, 128), jnp.float32),
          pltpu.VMEM_SHARED((NP, 128), jnp.float32),
          pltpu.SemaphoreType.DMA,
      ],
  )
  return sc_agg1, sc_agg4, sc_deg


IC1 = 16
IC4 = 32


def _gather_scatter_chunk(table_hbm, sv, dv, rows0, rows1, acc, gsem, ssem, ic):
  bufs = (rows0, rows1)
  gds = [pltpu.async_copy(table_hbm.at[sv.at[0]], bufs[0], gsem)]
  sds = []
  for r in range(ic):
    gds[r].wait()
    sds.append(pltpu.async_copy(bufs[r % 2], acc.at[dv.at[r]], ssem, add=True))
    if r >= 1:
      sds[r - 1].wait()
    if r + 1 < ic:
      gds.append(pltpu.async_copy(table_hbm.at[sv.at[r + 1]],
                                  bufs[(r + 1) % 2], gsem))
  sds[ic - 1].wait()


def _sc_agg1_body(table_hbm, src_hbm, dst_hbm, zeros_hbm, out_hbm,
                  sv, dv, rows0, rows1, acc, gsem, ssem):
  cid = lax.axis_index("c")
  sid = lax.axis_index("s")
  wid = cid * NS + sid
  nz = NODE_ROWS_PER_TILE
  pltpu.sync_copy(zeros_hbm.at[pl.ds(sid * nz, nz)], acc.at[pl.ds(sid * nz, nz)])
  nb = ROWS_PER_TILE_1
  base = wid * nb
  plsc.subcore_barrier()

  def outer(oc, carry):
    off = base + oc * IC1
    pltpu.sync_copy(src_hbm.at[pl.ds(off, IC1)], sv)
    pltpu.sync_copy(dst_hbm.at[pl.ds(off, IC1)], dv)
    _gather_scatter_chunk(table_hbm, sv, dv, rows0, rows1, acc, gsem, ssem, IC1)
    return carry

  lax.fori_loop(0, nb // IC1, outer, 0)
  plsc.subcore_barrier()
  pltpu.sync_copy(acc.at[pl.ds(sid * nz, nz)],
                  out_hbm.at[cid, pl.ds(sid * nz, nz)])


def _sc_deg_body(dst_hbm, zeros_hbm, ones_hbm, out_hbm, dv, ones_v, acc, ssem):
  cid = lax.axis_index("c")
  sid = lax.axis_index("s")
  wid = cid * NS + sid
  nz = NODE_ROWS_PER_TILE
  pltpu.sync_copy(zeros_hbm.at[pl.ds(sid * nz, nz)], acc.at[pl.ds(sid * nz, nz)])
  pltpu.sync_copy(ones_hbm, ones_v)
  nb = ROWS_PER_TILE_1
  base = wid * nb
  plsc.subcore_barrier()

  def outer(oc, carry):
    off = base + oc * IC1
    pltpu.sync_copy(dst_hbm.at[pl.ds(off, IC1)], dv)
    sds = [pltpu.async_copy(ones_v, acc.at[dv.at[r]], ssem, add=True)
           for r in range(IC1)]
    for d in sds:
      d.wait()
    return carry

  lax.fori_loop(0, nb // IC1, outer, 0)
  plsc.subcore_barrier()
  pltpu.sync_copy(acc.at[pl.ds(sid * nz, nz)],
                  out_hbm.at[cid, pl.ds(sid * nz, nz)])


def _sc_agg4_body(table_hbm, srcadj_hbm, dst_hbm, zeros_hbm, out_hbm,
                  sv, dv, rows0, rows1, acc, gsem, ssem):
  cid = lax.axis_index("c")
  sid = lax.axis_index("s")
  nz = NODE_ROWS_PER_TILE
  nb = ROWS_PER_TILE_4
  base = sid * nb
  for j in range(2):
    chunk = cid * 2 + j
    pltpu.sync_copy(zeros_hbm.at[pl.ds(sid * nz, nz)],
                    acc.at[pl.ds(sid * nz, nz)])
    plsc.subcore_barrier()

    def outer(oc, carry):
      off = base + oc * IC4
      pltpu.sync_copy(srcadj_hbm.at[chunk, pl.ds(off, IC4)], sv)
      pltpu.sync_copy(dst_hbm.at[pl.ds(off, IC4)], dv)
      _gather_scatter_chunk(table_hbm, sv, dv, rows0, rows1, acc, gsem, ssem, IC4)
      return carry

    lax.fori_loop(0, nb // IC4, outer, 0)
    plsc.subcore_barrier()
    pltpu.sync_copy(acc.at[pl.ds(sid * nz, nz)],
                    out_hbm.at[chunk, pl.ds(sid * nz, nz)])
    plsc.subcore_barrier()


NB = 1000
NBLK = N // NB


def _onehot(batch_blk):
  g = lax.broadcasted_iota(jnp.int32, (1, NG), 1)
  return (batch_blk == g).astype(jnp.float32)


def _moments_update(i, z, oh, s1_ref, s2_ref, cnt_ref):
  @pl.when(i == 0)
  def _():
    s1_ref[...] = jnp.zeros_like(s1_ref)
    s2_ref[...] = jnp.zeros_like(s2_ref)
    cnt_ref[...] = jnp.zeros_like(cnt_ref)

  dn = (((0,), (0,)), ((), ()))
  s1_ref[...] += lax.dot_general(oh, z, dn, preferred_element_type=jnp.float32, precision=lax.Precision.HIGHEST)
  s2_ref[...] += lax.dot_general(oh, z * z, dn, preferred_element_type=jnp.float32, precision=lax.Precision.HIGHEST)
  cnt_ref[...] += jnp.sum(oh, axis=0)[:, None]


def _alpha_beta(i, ms, w, b, s1_ref, s2_ref, cnt_ref, alpha_ref, beta_ref):
  @pl.when(i == NBLK - 1)
  def _():
    cnt = jnp.maximum(cnt_ref[:, 0:1], 1.0)
    m = s1_ref[...] / cnt
    var = s2_ref[...] / cnt - (2.0 * ms - ms * ms) * m * m
    std = jnp.sqrt(var + 1e-5)
    alpha = w / std
    alpha_ref[...] = alpha
    beta_ref[...] = b - alpha * m * ms


def _tc_a1_body(aggp, degp, x, batch_blk, Wl, bl, Wr, ms, w, b,
                z_ref, alpha_ref, beta_ref, s1_ref, s2_ref, cnt_ref):
  i = pl.program_id(0)
  deg = degp[0, :, 0:1] + degp[1, :, 0:1]
  invd = 1.0 / jnp.maximum(deg, 1.0)
  agg = (aggp[0] + aggp[1]) * invd
  z = (jnp.dot(agg, Wl[...], preferred_element_type=jnp.float32, precision=lax.Precision.HIGHEST)
       + jnp.dot(x[...], Wr[...], preferred_element_type=jnp.float32, precision=lax.Precision.HIGHEST)
       + bl[...])
  z_ref[...] = z
  oh = _onehot(batch_blk[...])
  _moments_update(i, z, oh, s1_ref, s2_ref, cnt_ref)
  _alpha_beta(i, ms[...], w[...], b[...], s1_ref, s2_ref, cnt_ref,
              alpha_ref, beta_ref)


def _tc_a_body(aggc, degp, hc, batch_blk, Wl, bl, Wr, ms, w, b,
               z_ref, alpha_ref, beta_ref, s1_ref, s2_ref, cnt_ref):
  i = pl.program_id(0)
  deg = degp[0, :, 0:1] + degp[1, :, 0:1]
  invd = 1.0 / jnp.maximum(deg, 1.0)
  z = bl[...]
  for c in range(4):
    z = z + jnp.dot(aggc[c] * invd, Wl[c], preferred_element_type=jnp.float32, precision=lax.Precision.HIGHEST)
    z = z + jnp.dot(hc[c], Wr[c], preferred_element_type=jnp.float32, precision=lax.Precision.HIGHEST)
  z_ref[...] = z
  oh = _onehot(batch_blk[...])
  _moments_update(i, z, oh, s1_ref, s2_ref, cnt_ref)
  _alpha_beta(i, ms[...], w[...], b[...], s1_ref, s2_ref, cnt_ref,
              alpha_ref, beta_ref)


def _tc_b_body(z, alpha, beta, batch_blk, hc_ref):
  oh = _onehot(batch_blk[...])
  a_rows = jnp.dot(oh, alpha[...], preferred_element_type=jnp.float32, precision=lax.Precision.HIGHEST)
  b_rows = jnp.dot(oh, beta[...], preferred_element_type=jnp.float32, precision=lax.Precision.HIGHEST)
  h = jnp.maximum(a_rows * z[...] + b_rows, 0.0)
  for c in range(4):
    hc_ref[c] = h[:, c * 128:(c + 1) * 128]


def _tc_b4_body(z, alpha, beta, batch_blk, Wl5, hc_ref, y5_ref):
  oh = _onehot(batch_blk[...])
  a_rows = jnp.dot(oh, alpha[...], preferred_element_type=jnp.float32, precision=lax.Precision.HIGHEST)
  b_rows = jnp.dot(oh, beta[...], preferred_element_type=jnp.float32, precision=lax.Precision.HIGHEST)
  h = jnp.maximum(a_rows * z[...] + b_rows, 0.0)
  for c in range(4):
    hc_ref[c] = h[:, c * 128:(c + 1) * 128]
  y5_ref[...] = jnp.dot(h, Wl5[...], preferred_element_type=jnp.float32, precision=lax.Precision.HIGHEST)


def _tc_final_body(y5p, degp, hc, bl, Wr, out_ref):
  deg = degp[0, :, 0:1] + degp[1, :, 0:1]
  invd = 1.0 / jnp.maximum(deg, 1.0)
  out = (y5p[0] + y5p[1]) * invd + bl[...]
  for c in range(4):
    out = out + jnp.dot(hc[c], Wr[c], preferred_element_type=jnp.float32, precision=lax.Precision.HIGHEST)
  out_ref[...] = out


def _row_spec(shape_prefix, block):
  pass


_spec_aggp = pl.BlockSpec((NC, NB, 128), lambda i: (0, i, 0))
_spec_degp = pl.BlockSpec((NC, NB, 128), lambda i: (0, i, 0))
_spec_x = pl.BlockSpec((NB, IN_F), lambda i: (i, 0))
_spec_batch = pl.BlockSpec((NB, 1), lambda i: (i, 0))
_spec_z = pl.BlockSpec((NB, HID), lambda i: (i, 0))
_spec_full2 = lambda a, bdim: pl.BlockSpec((a, bdim), lambda i: (0, 0))
_spec_hc = pl.BlockSpec((4, NB, 128), lambda i: (0, i, 0))
_spec_Wc = pl.BlockSpec((4, 128, HID), lambda i: (0, 0, 0))

_stat_scratch = [
    pltpu.VMEM((NG, HID), jnp.float32),
    pltpu.VMEM((NG, HID), jnp.float32),
    pltpu.VMEM((NG, 128), jnp.float32),
]
_ab_out_specs = [
    pl.BlockSpec((NB, HID), lambda i: (i, 0)),
    pl.BlockSpec((NG, HID), lambda i: (0, 0)),
    pl.BlockSpec((NG, HID), lambda i: (0, 0)),
]
_ab_out_shape = [
    jax.ShapeDtypeStruct((N, HID), jnp.float32),
    jax.ShapeDtypeStruct((NG, HID), jnp.float32),
    jax.ShapeDtypeStruct((NG, HID), jnp.float32),
]

_tc_a1 = pl.pallas_call(
    _tc_a1_body,
    grid=(NBLK,),
    in_specs=[
        _spec_aggp, _spec_degp, _spec_x, _spec_batch,
        _spec_full2(IN_F, HID), _spec_full2(1, HID), _spec_full2(IN_F, HID),
        _spec_full2(1, HID), _spec_full2(1, HID), _spec_full2(1, HID),
    ],
    out_specs=_ab_out_specs,
    out_shape=_ab_out_shape,
    scratch_shapes=_stat_scratch,
)

_tc_a = pl.pallas_call(
    _tc_a_body,
    grid=(NBLK,),
    in_specs=[
        pl.BlockSpec((4, NB, 128), lambda i: (0, i, 0)), _spec_degp, _spec_hc,
        _spec_batch,
        _spec_Wc, _spec_full2(1, HID), _spec_Wc,
        _spec_full2(1, HID), _spec_full2(1, HID), _spec_full2(1, HID),
    ],
    out_specs=_ab_out_specs,
    out_shape=_ab_out_shape,
    scratch_shapes=_stat_scratch,
)

_tc_b = pl.pallas_call(
    _tc_b_body,
    grid=(NBLK,),
    in_specs=[
        _spec_z, _spec_full2(NG, HID), _spec_full2(NG, HID), _spec_batch,
    ],
    out_specs=_spec_hc,
    out_shape=jax.ShapeDtypeStruct((4, N, 128), jnp.float32),
)

_tc_b4 = pl.pallas_call(
    _tc_b4_body,
    grid=(NBLK,),
    in_specs=[
        _spec_z, _spec_full2(NG, HID), _spec_full2(NG, HID), _spec_batch,
        _spec_full2(HID, 128),
    ],
    out_specs=[_spec_hc, pl.BlockSpec((NB, 128), lambda i: (i, 0))],
    out_shape=[
        jax.ShapeDtypeStruct((4, N, 128), jnp.float32),
        jax.ShapeDtypeStruct((N, 128), jnp.float32),
    ],
)

_tc_final = pl.pallas_call(
    _tc_final_body,
    grid=(NBLK,),
    in_specs=[
        _spec_aggp, _spec_degp, _spec_hc,
        pl.BlockSpec((1, 128), lambda i: (0, 0)),
        pl.BlockSpec((4, 128, 128), lambda i: (0, 0, 0)),
    ],
    out_specs=pl.BlockSpec((NB, 128), lambda i: (i, 0)),
    out_shape=jax.ShapeDtypeStruct((N, 128), jnp.float32),
)


def kernel(x, edge_index, batch,
           c1_Wl, c1_bl, c1_Wr, n1_w, n1_b, n1_ms,
           c2_Wl, c2_bl, c2_Wr, n2_w, n2_b, n2_ms,
           c3_Wl, c3_bl, c3_Wr, n3_w, n3_b, n3_ms,
           c4_Wl, c4_bl, c4_Wr, n4_w, n4_b, n4_ms,
           c5_Wl, c5_bl, c5_Wr):
  f32 = jnp.float32
  _sc_agg1, _sc_agg4, _sc_deg = _sc_kernels()
  src = edge_index[0]
  dst = edge_index[1]
  src_p = jnp.concatenate([src, jnp.zeros((EP - E,), jnp.int32)])
  pad_dst = N + jnp.arange(EP - E, dtype=jnp.int32) % (NP - N)
  dst_p = jnp.concatenate([dst, pad_dst])
  src2d = src_p.reshape(ROWS_TOTAL, LANES)
  dst2d = dst_p.reshape(ROWS_TOTAL, LANES)
  srcadj = (src_p[None, :] + (jnp.arange(4, dtype=jnp.int32) * N)[:, None]
            ).reshape(4, ROWS_TOTAL, LANES)
  zeros128 = jnp.zeros((NP, 128), f32)
  batch2d = batch.reshape(N, 1)

  degp = _sc_deg(dst2d, zeros128, jnp.ones((LANES, 128), f32))

  def row(v):
    return v.reshape(1, -1)

  p1 = _sc_agg1(x, src2d, dst2d, zeros128)
  z, al, be = _tc_a1(p1, degp, x, batch2d, c1_Wl, row(c1_bl), c1_Wr,
                     row(n1_ms), row(n1_w), row(n1_b))
  hc = _tc_b(z, al, be, batch2d)

  for (Wl, bl, Wr, msv, wv, bv) in (
      (c2_Wl, c2_bl, c2_Wr, n2_ms, n2_w, n2_b),
      (c3_Wl, c3_bl, c3_Wr, n3_ms, n3_w, n3_b),
  ):
    table = hc.reshape(4 * N, 128)
    s4 = _sc_agg4(table, srcadj, dst2d, zeros128)
    z, al, be = _tc_a(s4, degp, hc, batch2d,
                      Wl.reshape(4, 128, HID), row(bl), Wr.reshape(4, 128, HID),
                      row(msv), row(wv), row(bv))
    hc = _tc_b(z, al, be, batch2d)

  table = hc.reshape(4 * N, 128)
  s4 = _sc_agg4(table, srcadj, dst2d, zeros128)
  z, al, be = _tc_a(s4, degp, hc, batch2d,
                    c4_Wl.reshape(4, 128, HID), row(c4_bl),
                    c4_Wr.reshape(4, 128, HID),
                    row(n4_ms), row(n4_w), row(n4_b))
  Wl5p = jnp.pad(c5_Wl, ((0, 0), (0, 128 - OUT_F)))
  hc, y5 = _tc_b4(z, al, be, batch2d, Wl5p)

  p5 = _sc_agg1(y5, src2d, dst2d, zeros128)
  Wr5p = jnp.pad(c5_Wr, ((0, 0), (0, 128 - OUT_F))).reshape(4, 128, 128)
  bl5p = jnp.pad(c5_bl, (0, 128 - OUT_F)).reshape(1, 128)
  out = _tc_final(p5, degp, hc, bl5p, Wr5p)
  return out[:, :OUT_F]
